# baseline (speedup 1.0000x reference)
import jax
import jax.numpy as jnp
from jax import lax
from jax.experimental import pallas as pl
from jax.experimental.pallas import tpu as pltpu
from jax.experimental.pallas import tpu_sc as plsc

B = 4096
V = 100000
NC = 2
NS = 16
L = 16
NW = NC * NS
BPW = B // NW
NV = BPW // L
HV = 2


def _sc_body(inpt_hbm, tgt_hbm, out_hbm, tgt_v, seg_v, acc_v, *sems):
    wid = lax.axis_index("s") * NC + lax.axis_index("c")
    base = pl.multiple_of(wid * BPW, BPW)
    pltpu.sync_copy(tgt_hbm.at[pl.ds(base, BPW)], tgt_v)
    t16s = [tgt_v[pl.ds(i * L, L)] for i in range(NV)]
    copies = [
        pltpu.async_copy(
            inpt_hbm.at[t16s[i], pl.ds(base, BPW)],
            seg_v.at[pl.ds(i * L, L)],
            sems[i // HV],
        )
        for i in range(NV)
    ]
    lane_iota = lax.iota(jnp.int32, L)
    acc = jnp.zeros((L,), jnp.float32)
    for h in range(NV // HV):
        for c in copies[h * HV : (h + 1) * HV]:
            c.wait()
        for i in range(h * HV, (h + 1) * HV):
            diag = lane_iota + i * L
            vals = plsc.load_gather(seg_v, [diag, diag])
            acc = acc - jnp.where(t16s[i] > 0, vals, jnp.float32(0.0))
    acc_v[...] = acc
    pltpu.sync_copy(acc_v, out_hbm.at[wid])


@jax.jit
def kernel(input, target):
    tgt = target.reshape(B).astype(jnp.int32)
    mesh = plsc.VectorSubcoreMesh(core_axis_name="c", subcore_axis_name="s")
    parts = pl.kernel(
        _sc_body,
        out_type=jax.ShapeDtypeStruct((NW, L), jnp.float32),
        mesh=mesh,
        compiler_params=pltpu.CompilerParams(needs_layout_passes=False),
        scratch_types=[
            pltpu.VMEM((BPW,), jnp.int32),
            pltpu.VMEM((BPW, BPW), jnp.float32),
            pltpu.VMEM((L,), jnp.float32),
            pltpu.SemaphoreType.DMA,
            pltpu.SemaphoreType.DMA,
            pltpu.SemaphoreType.DMA,
            pltpu.SemaphoreType.DMA,
        ],
    )(input.T, tgt)
    return jnp.sum(parts)
